# baseline (speedup 1.0000x reference)
import jax
import jax.numpy as jnp
from jax import lax
from jax.experimental import pallas as pl
from jax.experimental.pallas import tpu as pltpu
from jax.experimental.pallas import tpu_sc as plsc

_NC = 2
_NS = 16
_NW = _NC * _NS
_BB = 128
_FP = 2


def _body(x_hbm, tab_hbm, out_hbm, xv, idx_v, rows_v,
          gsem0, gsem1, gsem2, ssem0, ssem1, ssem2):
    nf = x_hbm.shape[1]
    vocab = tab_hbm.shape[0] // nf
    wid = lax.axis_index("s") * _NC + lax.axis_index("c")
    nfu = nf // _FP
    nunit = (x_hbm.shape[0] // (_NW * _BB)) * nfu
    u0 = wid * nunit
    b0w = wid * (nunit // nfu) * _BB
    iota16 = lax.iota(jnp.int32, 16)

    def build_idx(u):
        @pl.when(lax.rem(u, nfu) == 0)
        def _restage():
            pltpu.sync_copy(
                x_hbm.at[pl.ds(b0w + lax.div(u, nfu) * _BB, _BB)], xv)

        for k in range(_FP):
            ll = u * _FP + k
            slot = lax.rem(u, 4) * _FP + k
            f = lax.rem(ll, nf)
            fv = jnp.full((16,), f, dtype=jnp.int32)
            off = f * vocab
            for j in range(_BB // 16):
                rows = j * 16 + iota16
                vals = plsc.load_gather(xv, [rows, fv])
                idx_v[slot, pl.ds(j * 16, 16)] = vals + off

    def gathers(u, buf, sem):
        for k in range(_FP):
            pltpu.async_copy(tab_hbm.at[idx_v.at[lax.rem(u, 4) * _FP + k]],
                             buf.at[:, pl.ds(k * 128, 128)], sem)

    def out_slab(u):
        ug = u0 + u
        nfu = nf // _FP
        bb = ug // nfu
        f0 = (ug - bb * nfu) * _FP
        return out_hbm.at[pl.ds(bb * _BB, _BB), pl.ds(f0 * 128, _FP * 128)]

    def wait_gathers(buf, sem):
        pltpu.make_async_copy(
            out_hbm.at[pl.ds(0, _BB), pl.ds(0, _FP * 128)], buf, sem).wait()

    def scatter(u, buf, sem):
        pltpu.async_copy(buf, out_slab(u), sem)

    def wait_scatter(buf, u, sem):
        pltpu.make_async_copy(buf, out_slab(u), sem).wait()

    bufs = [rows_v.at[0], rows_v.at[1], rows_v.at[2]]
    gs = [gsem0, gsem1, gsem2]
    ss = [ssem0, ssem1, ssem2]

    build_idx(0)
    gathers(0, bufs[0], gs[0])
    build_idx(1)
    build_idx(2)
    build_idx(3)
    wait_gathers(bufs[0], gs[0])
    scatter(0, bufs[0], ss[0])
    gathers(1, bufs[1], gs[1])
    wait_gathers(bufs[1], gs[1])
    scatter(1, bufs[1], ss[1])
    gathers(2, bufs[2], gs[2])

    @pl.loop(0, (nunit - 4) // 3)
    def _tri(p):
        for k in range(3):
            u = 3 * p + 2 + k
            b = (2 + k) % 3
            bn = (b + 1) % 3
            wait_gathers(bufs[b], gs[b])
            scatter(u, bufs[b], ss[b])
            wait_scatter(bufs[bn], u - 2, ss[bn])
            gathers(u + 1, bufs[bn], gs[bn])
            build_idx(u + 2)

    for u in (nunit - 2, nunit - 1):
        b = u % 3
        bn = (b + 1) % 3
        wait_gathers(bufs[b], gs[b])
        scatter(u, bufs[b], ss[b])
        if u + 1 < nunit:
            wait_scatter(bufs[bn], u - 2, ss[bn])
            gathers(u + 1, bufs[bn], gs[bn])
    for u in (nunit - 3, nunit - 2, nunit - 1):
        wait_scatter(bufs[u % 3], u, ss[u % 3])


def kernel(x, tables):
    b, f = x.shape
    nf, vocab, d = tables.shape
    bpw = b // _NW
    tab = tables.reshape(nf * vocab, d)
    out = pl.kernel(
        _body,
        out_type=jax.ShapeDtypeStruct((b, f * d), jnp.float32),
        mesh=plsc.VectorSubcoreMesh(core_axis_name="c", subcore_axis_name="s"),
        compiler_params=pltpu.CompilerParams(needs_layout_passes=False),
        scratch_types=[
            pltpu.VMEM((_BB, nf), jnp.int32),
            pltpu.VMEM((4 * _FP, _BB), jnp.int32),
            pltpu.VMEM((3, _BB, _FP * d), jnp.float32),
        ] + [pltpu.SemaphoreType.DMA] * 6,
    )(x.astype(jnp.int32), tab)
    return out


# speedup vs baseline: 1.0285x; 1.0285x over previous
import jax
import jax.numpy as jnp
from jax import lax
from jax.experimental import pallas as pl
from jax.experimental.pallas import tpu as pltpu
from jax.experimental.pallas import tpu_sc as plsc

_NC = 2
_NS = 16
_NW = _NC * _NS
_BB = 128
_FP = 2


def _body(idx_hbm, tab_hbm, out_hbm, idx_v, rows_v,
          gsem0, gsem1, gsem2, ssem0, ssem1, ssem2):
    nf = out_hbm.shape[1] // 128
    wid = lax.axis_index("s") * _NC + lax.axis_index("c")
    nunit = idx_v.shape[0] // _FP
    u0 = wid * nunit
    pltpu.sync_copy(idx_hbm.at[wid], idx_v)

    def gathers(u, buf, sem):
        for k in range(_FP):
            pltpu.async_copy(tab_hbm.at[idx_v.at[u * _FP + k]],
                             buf.at[:, pl.ds(k * 128, 128)], sem)

    def out_slab(u):
        ug = u0 + u
        nfu = nf // _FP
        bb = ug // nfu
        f0 = (ug - bb * nfu) * _FP
        return out_hbm.at[pl.ds(bb * _BB, _BB), pl.ds(f0 * 128, _FP * 128)]

    def wait_gathers(buf, sem):
        pltpu.make_async_copy(
            out_hbm.at[pl.ds(0, _BB), pl.ds(0, _FP * 128)], buf, sem).wait()

    def scatter(u, buf, sem):
        pltpu.async_copy(buf, out_slab(u), sem)

    def wait_scatter(buf, u, sem):
        pltpu.make_async_copy(buf, out_slab(u), sem).wait()

    bufs = [rows_v.at[0], rows_v.at[1], rows_v.at[2]]
    gs = [gsem0, gsem1, gsem2]
    ss = [ssem0, ssem1, ssem2]

    gathers(0, bufs[0], gs[0])
    wait_gathers(bufs[0], gs[0])
    scatter(0, bufs[0], ss[0])
    gathers(1, bufs[1], gs[1])
    wait_gathers(bufs[1], gs[1])
    scatter(1, bufs[1], ss[1])
    gathers(2, bufs[2], gs[2])

    @pl.loop(0, (nunit - 4) // 3)
    def _tri(p):
        for k in range(3):
            u = 3 * p + 2 + k
            b = (2 + k) % 3
            bn = (b + 1) % 3
            wait_gathers(bufs[b], gs[b])
            scatter(u, bufs[b], ss[b])
            wait_scatter(bufs[bn], u - 2, ss[bn])
            gathers(u + 1, bufs[bn], gs[bn])

    for u in (nunit - 2, nunit - 1):
        b = u % 3
        bn = (b + 1) % 3
        wait_gathers(bufs[b], gs[b])
        scatter(u, bufs[b], ss[b])
        if u + 1 < nunit:
            wait_scatter(bufs[bn], u - 2, ss[bn])
            gathers(u + 1, bufs[bn], gs[bn])
    for u in (nunit - 3, nunit - 2, nunit - 1):
        wait_scatter(bufs[u % 3], u, ss[u % 3])


def kernel(x, tables):
    b, f = x.shape
    nf, vocab, d = tables.shape
    nbb = b // _BB
    rows_per_w = (nbb * nf * _BB) // _NW
    lists_per_w = rows_per_w // _BB
    flat_idx = (x.astype(jnp.int32)
                + jnp.arange(nf, dtype=jnp.int32)[None, :] * vocab)
    flat_idx = flat_idx.reshape(nbb, _BB, nf).transpose(0, 2, 1)
    flat_idx = flat_idx.reshape(_NW, lists_per_w, _BB)
    tab = tables.reshape(nf * vocab, d)
    out = pl.kernel(
        _body,
        out_type=jax.ShapeDtypeStruct((b, f * d), jnp.float32),
        mesh=plsc.VectorSubcoreMesh(core_axis_name="c", subcore_axis_name="s"),
        compiler_params=pltpu.CompilerParams(use_tc_tiling_on_sc=True),
        scratch_types=[
            pltpu.VMEM((lists_per_w, _BB), jnp.int32),
            pltpu.VMEM((3, _BB, _FP * d), jnp.float32),
        ] + [pltpu.SemaphoreType.DMA] * 6,
    )(flat_idx, tab)
    return out
